# baseline (speedup 1.0000x reference)
import functools
import math

import jax
import jax.numpy as jnp
from jax.experimental import pallas as pl
from jax.experimental.pallas import tpu as pltpu

H = 768
HEADS = 12
DH = H // HEADS
FFN = 4 * H
FC_PAD = 128
NUM_CLASSES = 4
LN_EPS = 1e-12
_VMEM_LIMIT = 48 * 1024 * 1024


def _gelu(x):
    c = math.sqrt(2.0 / math.pi)
    return 0.5 * x * (1.0 + jnp.tanh(c * (x + 0.044715 * x * x * x)))


def _layernorm(y, g, b):
    mean = jnp.mean(y, axis=-1, keepdims=True)
    yc = y - mean
    var = jnp.mean(yc * yc, axis=-1, keepdims=True)
    return yc * jax.lax.rsqrt(var + LN_EPS) * g + b


def _enc_kernel(emb_ref, am_ref, eg_ref, eb_ref,
                wqkv_ref, bqkv_ref, wo_ref, bo_ref, g1_ref, bt1_ref,
                w1_ref, b1_ref, w2_ref, b2_ref, g2_ref, bt2_ref,
                pw_ref, pb_ref, fw_ref, fb_ref,
                pooled_ref, logits_ref,
                h_s, qkv_s, ctx_s, *, seq_len, nseq):
    l = pl.program_id(1)
    cm = h_s.shape[0]

    @pl.when(l == 0)
    def _():
        h_s[...] = _layernorm(emb_ref[...], eg_ref[...], eb_ref[...])

    x = h_s[...]

    qkv = jnp.dot(x.astype(jnp.bfloat16), wqkv_ref[...],
                  preferred_element_type=jnp.float32) + bqkv_ref[...]
    qkv_s[...] = qkv.astype(jnp.bfloat16)

    row_b = jax.lax.broadcasted_iota(jnp.int32, (cm, cm), 0) // seq_len
    col_b = jax.lax.broadcasted_iota(jnp.int32, (cm, cm), 1) // seq_len
    keep = (row_b == col_b) & (am_ref[...] > 0.5)
    bias = jnp.where(keep, 0.0, -1e9).astype(jnp.float32)

    scale = 1.0 / math.sqrt(DH)
    for hh in range(HEADS):
        q = qkv_s[:, hh * DH:(hh + 1) * DH]
        k = qkv_s[:, H + hh * DH:H + (hh + 1) * DH]
        v = qkv_s[:, 2 * H + hh * DH:2 * H + (hh + 1) * DH]

        s = jnp.einsum("qd,kd->qk", q, k,
                       preferred_element_type=jnp.float32) * scale + bias
        mx = jnp.max(s, axis=-1, keepdims=True)
        p = jnp.exp(s - mx)
        p = p * pl.reciprocal(jnp.sum(p, axis=-1, keepdims=True), approx=True)
        ctx = jnp.dot(p.astype(jnp.bfloat16), v,
                      preferred_element_type=jnp.float32)
        ctx_s[:, hh * DH:(hh + 1) * DH] = ctx.astype(jnp.bfloat16)

    attn = jnp.dot(ctx_s[...], wo_ref[...],
                   preferred_element_type=jnp.float32)
    y = attn + bo_ref[...] + x
    h1 = _layernorm(y, g1_ref[...], bt1_ref[...])

    ff = jnp.dot(h1.astype(jnp.bfloat16), w1_ref[...],
                 preferred_element_type=jnp.float32) + b1_ref[...]
    ff = _gelu(ff)
    y2 = jnp.dot(ff.astype(jnp.bfloat16), w2_ref[...],
                 preferred_element_type=jnp.float32) + b2_ref[...] + h1
    h2 = _layernorm(y2, g2_ref[...], bt2_ref[...])
    h_s[...] = h2

    @pl.when(l == pl.num_programs(1) - 1)
    def _():
        rows = [h2[i * seq_len:i * seq_len + 1, :] for i in range(nseq)]
        rows += [h2[0:1, :]] * (8 - nseq)
        cls = jnp.concatenate(rows, axis=0)
        pooled = jnp.tanh(jnp.dot(cls.astype(jnp.bfloat16), pw_ref[...],
                                  preferred_element_type=jnp.float32)
                          + pb_ref[...])
        logits = jnp.dot(pooled.astype(jnp.bfloat16), fw_ref[...],
                         preferred_element_type=jnp.float32) + fb_ref[...]
        pooled_ref[...] = pooled
        logits_ref[...] = logits


def kernel(word_emb, pos_emb, type_emb, emb_ln_g, emb_ln_b, pool_w, pool_b,
           fc_w_pad, fc_b_pad, enc_wqkv, enc_bqkv, enc_wo, enc_bo,
           enc_ln1_g, enc_ln1_b, enc_w1, enc_b1, enc_w2, enc_b2,
           enc_ln2_g, enc_ln2_b, input_ids, attention_mask):
    Bq, Sq = input_ids.shape
    M = Bq * Sq
    L = enc_wqkv.shape[0]
    nchunk = 2 if Bq % 2 == 0 else 1
    nseq = Bq // nchunk
    cm = nseq * Sq

    emb = (word_emb[input_ids] + pos_emb[:Sq][None, :, :]
           + type_emb[0][None, None, :]).reshape(M, H).astype(jnp.float32)
    am = attention_mask.astype(jnp.float32).reshape(nchunk, 1, cm)

    def _const(shape):
        return pl.BlockSpec(shape, lambda c, l, _n=len(shape): (0,) * _n)

    def _layer(shape):
        return pl.BlockSpec((None,) + shape,
                            lambda c, l, _n=len(shape): (l,) + (0,) * _n)

    kern = functools.partial(_enc_kernel, seq_len=Sq, nseq=nseq)
    pooled_pad, logits_pad = pl.pallas_call(
        kern,
        out_shape=(jax.ShapeDtypeStruct((nchunk, 8, H), jnp.float32),
                   jax.ShapeDtypeStruct((nchunk, 8, FC_PAD), jnp.float32)),
        grid_spec=pltpu.PrefetchScalarGridSpec(
            num_scalar_prefetch=0,
            grid=(nchunk, L),
            in_specs=[
                pl.BlockSpec((cm, H), lambda c, l: (c, 0)),
                pl.BlockSpec((None, 1, cm), lambda c, l: (c, 0, 0)),
                _const((1, H)), _const((1, H)),
                _layer((H, 3 * H)), _layer((1, 3 * H)),
                _layer((H, H)), _layer((1, H)),
                _layer((1, H)), _layer((1, H)),
                _layer((H, FFN)), _layer((1, FFN)),
                _layer((FFN, H)), _layer((1, H)),
                _layer((1, H)), _layer((1, H)),
                _const((H, H)), _const((1, H)),
                _const((H, FC_PAD)), _const((1, FC_PAD)),
            ],
            out_specs=[
                pl.BlockSpec((None, 8, H), lambda c, l: (c, 0, 0)),
                pl.BlockSpec((None, 8, FC_PAD), lambda c, l: (c, 0, 0)),
            ],
            scratch_shapes=[
                pltpu.VMEM((cm, H), jnp.float32),
                pltpu.VMEM((cm, 3 * H), jnp.bfloat16),
                pltpu.VMEM((cm, H), jnp.bfloat16),
            ],
        ),
        compiler_params=pltpu.CompilerParams(
            dimension_semantics=("arbitrary", "arbitrary"),
            vmem_limit_bytes=_VMEM_LIMIT),
    )(emb, am, emb_ln_g.reshape(1, H), emb_ln_b.reshape(1, H),
      enc_wqkv, enc_bqkv, enc_wo, enc_bo, enc_ln1_g, enc_ln1_b,
      enc_w1, enc_b1, enc_w2, enc_b2, enc_ln2_g, enc_ln2_b,
      pool_w, pool_b.reshape(1, H), fc_w_pad, fc_b_pad.reshape(1, FC_PAD))

    pooled = pooled_pad[:, :nseq, :].reshape(Bq, H)
    logits = logits_pad[:, :nseq, :NUM_CLASSES].reshape(Bq, NUM_CLASSES)
    return logits, pooled


# speedup vs baseline: 1.2792x; 1.2792x over previous
import functools
import math

import jax
import jax.numpy as jnp
from jax.experimental import pallas as pl
from jax.experimental.pallas import tpu as pltpu

H = 768
HEADS = 12
DH = H // HEADS
FFN = 4 * H
FC_PAD = 128
NUM_CLASSES = 4
LN_EPS = 1e-12
_VMEM_LIMIT = 48 * 1024 * 1024


def _gelu(x):
    c = math.sqrt(2.0 / math.pi)
    return 0.5 * x * (1.0 + jnp.tanh(c * (x + 0.044715 * x * x * x)))


def _layernorm(y, g, b):
    mean = jnp.mean(y, axis=-1, keepdims=True)
    yc = y - mean
    var = jnp.mean(yc * yc, axis=-1, keepdims=True)
    return yc * jax.lax.rsqrt(var + LN_EPS) * g + b


def _softmax_rows(s):
    mx = jnp.max(s, axis=-1, keepdims=True)
    p = jnp.exp(s - mx)
    return p * pl.reciprocal(jnp.sum(p, axis=-1, keepdims=True), approx=True)


def _enc_kernel(emb_ref, am_ref, eg_ref, eb_ref,
                wqkv_ref, bqkv_ref, wo_ref, bo_ref, g1_ref, bt1_ref,
                w1_ref, b1_ref, w2_ref, b2_ref, g2_ref, bt2_ref,
                pw_ref, pb_ref, fw_ref, fb_ref,
                pooled_ref, logits_ref,
                h_s, qkv_s, ctx_s, *, seq_len, nseq, bw):
    l = pl.program_id(0)
    m = h_s.shape[0]
    nblk = m // bw
    scale = 1.0 / math.sqrt(DH)

    @pl.when(l == 0)
    def _():
        h_s[...] = _layernorm(emb_ref[...], eg_ref[...], eb_ref[...])

    x = h_s[...]

    @pl.when(l < pl.num_programs(0) - 1)
    def _():
        qkv = jnp.dot(x.astype(jnp.bfloat16), wqkv_ref[...],
                      preferred_element_type=jnp.float32) + bqkv_ref[...]
        qkv_s[...] = qkv.astype(jnp.bfloat16)

        row_b = jax.lax.broadcasted_iota(jnp.int32, (bw, bw), 0) // seq_len
        col_b = jax.lax.broadcasted_iota(jnp.int32, (bw, bw), 1) // seq_len
        same_seq = row_b == col_b
        for blk in range(nblk):
            r = slice(blk * bw, (blk + 1) * bw)
            keep = same_seq & (am_ref[:, r] > 0.5)
            bias = jnp.where(keep, 0.0, -1e9).astype(jnp.float32)
            for hh in range(HEADS):
                q = qkv_s[r, hh * DH:(hh + 1) * DH]
                k = qkv_s[r, H + hh * DH:H + (hh + 1) * DH]
                v = qkv_s[r, 2 * H + hh * DH:2 * H + (hh + 1) * DH]
                s = jnp.einsum("qd,kd->qk", q, k,
                               preferred_element_type=jnp.float32) * scale + bias
                p = _softmax_rows(s)
                ctx = jnp.dot(p.astype(jnp.bfloat16), v,
                              preferred_element_type=jnp.float32)
                ctx_s[r, hh * DH:(hh + 1) * DH] = ctx.astype(jnp.bfloat16)

        attn = jnp.dot(ctx_s[...], wo_ref[...],
                       preferred_element_type=jnp.float32)
        y = attn + bo_ref[...] + x
        h1 = _layernorm(y, g1_ref[...], bt1_ref[...])

        ff = jnp.dot(h1.astype(jnp.bfloat16), w1_ref[...],
                     preferred_element_type=jnp.float32) + b1_ref[...]
        ff = _gelu(ff)
        y2 = jnp.dot(ff.astype(jnp.bfloat16), w2_ref[...],
                     preferred_element_type=jnp.float32) + b2_ref[...] + h1
        h_s[...] = _layernorm(y2, g2_ref[...], bt2_ref[...])

    @pl.when(l == pl.num_programs(0) - 1)
    def _():
        xb = x.astype(jnp.bfloat16)
        kv = (jnp.dot(xb, wqkv_ref[:, H:],
                      preferred_element_type=jnp.float32)
              + bqkv_ref[:, H:]).astype(jnp.bfloat16)

        cls_x = jnp.concatenate(
            [x[i * seq_len:i * seq_len + 1, :] for i in range(nseq)],
            axis=0)
        q = jnp.dot(cls_x.astype(jnp.bfloat16), wqkv_ref[:, :H],
                    preferred_element_type=jnp.float32) + bqkv_ref[:, :H]
        qb = q.astype(jnp.bfloat16)

        row_b = jax.lax.broadcasted_iota(jnp.int32, (nseq, m), 0)
        col_b = jax.lax.broadcasted_iota(jnp.int32, (nseq, m), 1) // seq_len
        keep = (row_b == col_b) & (am_ref[...] > 0.5)
        bias = jnp.where(keep, 0.0, -1e9).astype(jnp.float32)

        ctxs = []
        for hh in range(HEADS):
            qh = qb[:, hh * DH:(hh + 1) * DH]
            kh = kv[:, hh * DH:(hh + 1) * DH]
            vh = kv[:, H + hh * DH:H + (hh + 1) * DH]
            s = jnp.einsum("qd,kd->qk", qh, kh,
                           preferred_element_type=jnp.float32) * scale + bias
            p = _softmax_rows(s)
            ctxs.append(jnp.dot(p.astype(jnp.bfloat16), vh,
                                preferred_element_type=jnp.float32))
        ctx = jnp.concatenate(ctxs, axis=-1)

        attn = jnp.dot(ctx.astype(jnp.bfloat16), wo_ref[...],
                       preferred_element_type=jnp.float32)
        y = attn + bo_ref[...] + cls_x
        h1 = _layernorm(y, g1_ref[...], bt1_ref[...])

        ff = jnp.dot(h1.astype(jnp.bfloat16), w1_ref[...],
                     preferred_element_type=jnp.float32) + b1_ref[...]
        ff = _gelu(ff)
        y2 = jnp.dot(ff.astype(jnp.bfloat16), w2_ref[...],
                     preferred_element_type=jnp.float32) + b2_ref[...] + h1
        h2 = _layernorm(y2, g2_ref[...], bt2_ref[...])

        pooled = jnp.tanh(jnp.dot(h2.astype(jnp.bfloat16), pw_ref[...],
                                  preferred_element_type=jnp.float32)
                          + pb_ref[...])
        logits = jnp.dot(pooled.astype(jnp.bfloat16), fw_ref[...],
                         preferred_element_type=jnp.float32) + fb_ref[...]
        pooled_ref[...] = pooled
        logits_ref[...] = logits


def kernel(word_emb, pos_emb, type_emb, emb_ln_g, emb_ln_b, pool_w, pool_b,
           fc_w_pad, fc_b_pad, enc_wqkv, enc_bqkv, enc_wo, enc_bo,
           enc_ln1_g, enc_ln1_b, enc_w1, enc_b1, enc_w2, enc_b2,
           enc_ln2_g, enc_ln2_b, input_ids, attention_mask):
    Bq, Sq = input_ids.shape
    M = Bq * Sq
    L = enc_wqkv.shape[0]
    bw = 4 * Sq if Bq % 4 == 0 else Sq

    emb = (word_emb[input_ids] + pos_emb[:Sq][None, :, :]
           + type_emb[0][None, None, :]).reshape(M, H).astype(jnp.float32)
    am = attention_mask.astype(jnp.float32).reshape(1, M)

    def _const(shape):
        return pl.BlockSpec(shape, lambda l, _n=len(shape): (0,) * _n)

    def _layer(shape):
        return pl.BlockSpec((None,) + shape,
                            lambda l, _n=len(shape): (l,) + (0,) * _n)

    kern = functools.partial(_enc_kernel, seq_len=Sq, nseq=Bq, bw=bw)
    pooled, logits_pad = pl.pallas_call(
        kern,
        out_shape=(jax.ShapeDtypeStruct((Bq, H), jnp.float32),
                   jax.ShapeDtypeStruct((Bq, FC_PAD), jnp.float32)),
        grid_spec=pltpu.PrefetchScalarGridSpec(
            num_scalar_prefetch=0,
            grid=(L,),
            in_specs=[
                _const((M, H)),
                _const((1, M)),
                _const((1, H)), _const((1, H)),
                _layer((H, 3 * H)), _layer((1, 3 * H)),
                _layer((H, H)), _layer((1, H)),
                _layer((1, H)), _layer((1, H)),
                _layer((H, FFN)), _layer((1, FFN)),
                _layer((FFN, H)), _layer((1, H)),
                _layer((1, H)), _layer((1, H)),
                _const((H, H)), _const((1, H)),
                _const((H, FC_PAD)), _const((1, FC_PAD)),
            ],
            out_specs=[
                pl.BlockSpec((Bq, H), lambda l: (0, 0)),
                pl.BlockSpec((Bq, FC_PAD), lambda l: (0, 0)),
            ],
            scratch_shapes=[
                pltpu.VMEM((M, H), jnp.float32),
                pltpu.VMEM((M, 3 * H), jnp.bfloat16),
                pltpu.VMEM((M, H), jnp.bfloat16),
            ],
        ),
        compiler_params=pltpu.CompilerParams(
            dimension_semantics=("arbitrary",),
            vmem_limit_bytes=_VMEM_LIMIT),
    )(emb, am, emb_ln_g.reshape(1, H), emb_ln_b.reshape(1, H),
      enc_wqkv, enc_bqkv, enc_wo, enc_bo, enc_ln1_g, enc_ln1_b,
      enc_w1, enc_b1, enc_w2, enc_b2, enc_ln2_g, enc_ln2_b,
      pool_w, pool_b.reshape(1, H), fc_w_pad, fc_b_pad.reshape(1, FC_PAD))

    logits = logits_pad[:, :NUM_CLASSES]
    return logits, pooled


# speedup vs baseline: 1.4768x; 1.1545x over previous
import functools
import math

import jax
import jax.numpy as jnp
from jax.experimental import pallas as pl
from jax.experimental.pallas import tpu as pltpu

H = 768
HEADS = 12
DH = H // HEADS
FFN = 4 * H
FC_PAD = 128
NUM_CLASSES = 4
LN_EPS = 1e-12
_VMEM_LIMIT = 48 * 1024 * 1024


def _gelu(x):
    c = math.sqrt(2.0 / math.pi)
    return 0.5 * x * (1.0 + jnp.tanh(c * (x + 0.044715 * x * x * x)))


def _layernorm(y, g, b):
    mean = jnp.mean(y, axis=-1, keepdims=True)
    yc = y - mean
    var = jnp.mean(yc * yc, axis=-1, keepdims=True)
    return yc * jax.lax.rsqrt(var + LN_EPS) * g + b


def _softmax_rows(s):
    mx = jnp.max(s, axis=-1, keepdims=True)
    p = jnp.exp(s - mx)
    return p * pl.reciprocal(jnp.sum(p, axis=-1, keepdims=True), approx=True)


def _mha(qkv, bias):
    qb = qkv[:, :H].astype(jnp.bfloat16)
    kvb = qkv[:, H:].astype(jnp.bfloat16)
    ctxs = []
    for hh in range(HEADS):
        q = qb[:, hh * DH:(hh + 1) * DH]
        k = kvb[:, hh * DH:(hh + 1) * DH]
        v = kvb[:, H + hh * DH:H + (hh + 1) * DH]
        s = jnp.einsum("qd,kd->qk", q, k,
                       preferred_element_type=jnp.float32) + bias
        p = _softmax_rows(s)
        ctxs.append(jnp.dot(p.astype(jnp.bfloat16), v,
                            preferred_element_type=jnp.float32))
    return jnp.concatenate(ctxs, axis=-1)


def _enc_kernel(emb_ref, am_ref, eg_ref, eb_ref,
                wqkv_ref, bqkv_ref, wo_ref, bo_ref, g1_ref, bt1_ref,
                w1_ref, b1_ref, w2_ref, b2_ref, g2_ref, bt2_ref,
                pw_ref, pb_ref, fw_ref, fb_ref,
                pooled_ref, logits_ref,
                h_s, *, seq_len, nseq, bw):
    l = pl.program_id(0)
    m = h_s.shape[0]
    nblk = m // bw
    scale = 1.0 / math.sqrt(DH)
    qscale = jnp.concatenate(
        [jnp.full((1, H), scale, jnp.float32),
         jnp.ones((1, 2 * H), jnp.float32)], axis=-1)

    @pl.when(l == 0)
    def _():
        h_s[...] = _layernorm(emb_ref[...], eg_ref[...], eb_ref[...])

    @pl.when(l < pl.num_programs(0) - 1)
    def _():
        row_b = jax.lax.broadcasted_iota(jnp.int32, (bw, bw), 0) // seq_len
        col_b = jax.lax.broadcasted_iota(jnp.int32, (bw, bw), 1) // seq_len
        same_seq = row_b == col_b
        for blk in range(nblk):
            r = slice(blk * bw, (blk + 1) * bw)
            x = h_s[r, :]
            qkv = (jnp.dot(x.astype(jnp.bfloat16), wqkv_ref[...],
                           preferred_element_type=jnp.float32)
                   + bqkv_ref[...]) * qscale
            keep = same_seq & (am_ref[:, r] > 0.5)
            bias = jnp.where(keep, 0.0, -1e9).astype(jnp.float32)
            ctx = _mha(qkv, bias)
            attn = jnp.dot(ctx.astype(jnp.bfloat16), wo_ref[...],
                           preferred_element_type=jnp.float32)
            h1 = _layernorm(attn + bo_ref[...] + x, g1_ref[...], bt1_ref[...])
            ff = jnp.dot(h1.astype(jnp.bfloat16), w1_ref[...],
                         preferred_element_type=jnp.float32) + b1_ref[...]
            ff = _gelu(ff)
            y2 = jnp.dot(ff.astype(jnp.bfloat16), w2_ref[...],
                         preferred_element_type=jnp.float32) + b2_ref[...] + h1
            h_s[r, :] = _layernorm(y2, g2_ref[...], bt2_ref[...])

    @pl.when(l == pl.num_programs(0) - 1)
    def _():
        x = h_s[...]
        kv = (jnp.dot(x.astype(jnp.bfloat16), wqkv_ref[:, H:],
                      preferred_element_type=jnp.float32)
              + bqkv_ref[:, H:]).astype(jnp.bfloat16)

        cls_x = jnp.concatenate(
            [x[i * seq_len:i * seq_len + 1, :] for i in range(nseq)],
            axis=0)
        q = (jnp.dot(cls_x.astype(jnp.bfloat16), wqkv_ref[:, :H],
                     preferred_element_type=jnp.float32)
             + bqkv_ref[:, :H]) * scale

        nr = HEADS * nseq
        qtile = jnp.concatenate([q] * HEADS, axis=0)
        rowh = jax.lax.broadcasted_iota(jnp.int32, (nr, H), 0) // nseq
        colh = jax.lax.broadcasted_iota(jnp.int32, (nr, H), 1) // DH
        qstack = jnp.where(rowh == colh, qtile, 0.0).astype(jnp.bfloat16)

        s = jnp.einsum("qd,kd->qk", qstack, kv[:, :H],
                       preferred_element_type=jnp.float32)
        rowi = jax.lax.broadcasted_iota(jnp.int32, (nr, m), 0) % nseq
        colb = jax.lax.broadcasted_iota(jnp.int32, (nr, m), 1) // seq_len
        keep = (rowi == colb) & (am_ref[...] > 0.5)
        bias = jnp.where(keep, 0.0, -1e9).astype(jnp.float32)
        p = _softmax_rows(s + bias)
        ctx_all = jnp.dot(p.astype(jnp.bfloat16), kv[:, H:],
                          preferred_element_type=jnp.float32)
        ctx = jnp.concatenate(
            [ctx_all[hh * nseq:(hh + 1) * nseq, hh * DH:(hh + 1) * DH]
             for hh in range(HEADS)], axis=-1)

        attn = jnp.dot(ctx.astype(jnp.bfloat16), wo_ref[...],
                       preferred_element_type=jnp.float32)
        h1 = _layernorm(attn + bo_ref[...] + cls_x, g1_ref[...], bt1_ref[...])
        ff = jnp.dot(h1.astype(jnp.bfloat16), w1_ref[...],
                     preferred_element_type=jnp.float32) + b1_ref[...]
        ff = _gelu(ff)
        y2 = jnp.dot(ff.astype(jnp.bfloat16), w2_ref[...],
                     preferred_element_type=jnp.float32) + b2_ref[...] + h1
        h2 = _layernorm(y2, g2_ref[...], bt2_ref[...])

        pooled = jnp.tanh(jnp.dot(h2.astype(jnp.bfloat16), pw_ref[...],
                                  preferred_element_type=jnp.float32)
                          + pb_ref[...])
        logits = jnp.dot(pooled.astype(jnp.bfloat16), fw_ref[...],
                         preferred_element_type=jnp.float32) + fb_ref[...]
        pooled_ref[...] = pooled
        logits_ref[...] = logits


def kernel(word_emb, pos_emb, type_emb, emb_ln_g, emb_ln_b, pool_w, pool_b,
           fc_w_pad, fc_b_pad, enc_wqkv, enc_bqkv, enc_wo, enc_bo,
           enc_ln1_g, enc_ln1_b, enc_w1, enc_b1, enc_w2, enc_b2,
           enc_ln2_g, enc_ln2_b, input_ids, attention_mask):
    Bq, Sq = input_ids.shape
    M = Bq * Sq
    L = enc_wqkv.shape[0]
    bw = Sq
    while bw < 256 and M % (2 * bw) == 0:
        bw *= 2

    emb = (word_emb[input_ids] + pos_emb[:Sq][None, :, :]
           + type_emb[0][None, None, :]).reshape(M, H).astype(jnp.float32)
    am = attention_mask.astype(jnp.float32).reshape(1, M)

    def _const(shape):
        return pl.BlockSpec(shape, lambda l, _n=len(shape): (0,) * _n)

    def _layer(shape):
        return pl.BlockSpec((None,) + shape,
                            lambda l, _n=len(shape): (l,) + (0,) * _n)

    kern = functools.partial(_enc_kernel, seq_len=Sq, nseq=Bq, bw=bw)
    pooled, logits_pad = pl.pallas_call(
        kern,
        out_shape=(jax.ShapeDtypeStruct((Bq, H), jnp.float32),
                   jax.ShapeDtypeStruct((Bq, FC_PAD), jnp.float32)),
        grid_spec=pltpu.PrefetchScalarGridSpec(
            num_scalar_prefetch=0,
            grid=(L,),
            in_specs=[
                _const((M, H)),
                _const((1, M)),
                _const((1, H)), _const((1, H)),
                _layer((H, 3 * H)), _layer((1, 3 * H)),
                _layer((H, H)), _layer((1, H)),
                _layer((1, H)), _layer((1, H)),
                _layer((H, FFN)), _layer((1, FFN)),
                _layer((FFN, H)), _layer((1, H)),
                _layer((1, H)), _layer((1, H)),
                _const((H, H)), _const((1, H)),
                _const((H, FC_PAD)), _const((1, FC_PAD)),
            ],
            out_specs=[
                pl.BlockSpec((Bq, H), lambda l: (0, 0)),
                pl.BlockSpec((Bq, FC_PAD), lambda l: (0, 0)),
            ],
            scratch_shapes=[
                pltpu.VMEM((M, H), jnp.float32),
            ],
        ),
        compiler_params=pltpu.CompilerParams(
            dimension_semantics=("arbitrary",),
            vmem_limit_bytes=_VMEM_LIMIT),
    )(emb, am, emb_ln_g.reshape(1, H), emb_ln_b.reshape(1, H),
      enc_wqkv, enc_bqkv, enc_wo, enc_bo, enc_ln1_g, enc_ln1_b,
      enc_w1, enc_b1, enc_w2, enc_b2, enc_ln2_g, enc_ln2_b,
      pool_w, pool_b.reshape(1, H), fc_w_pad, fc_b_pad.reshape(1, FC_PAD))

    logits = logits_pad[:, :NUM_CLASSES]
    return logits, pooled
